# baseline (speedup 1.0000x reference)
import functools

import jax
import jax.numpy as jnp
from jax import lax
from jax.experimental import pallas as pl
from jax.experimental.pallas import tpu as pltpu
from jax.experimental.pallas import tpu_sc as plsc

NUM_UNITS = 64
TOP_K = 8
NUM_LABELS = 1000
HIDDEN = 128
INCH = 768
HW = 196
BATCH = 512
BBLK = 16

NW = 32
ROWS_PER_W = BATCH // NW


def _tc_body(x_ref, w1_ref, b1_ref, w2_ref, b2_ref, w3_ref, b3_ref,
             w4_ref, b4_ref, wu_ref, bu_ref, wc_ref, bc_ref, noise_ref,
             scores_ref, logits_ref, l2_ref):
    f32 = jnp.float32
    dn = (((1,), (1,)), ((), ()))
    pooled = jnp.mean(x_ref[...], axis=2)
    h = jax.nn.relu(lax.dot_general(pooled, w1_ref[...], dn,
                                    preferred_element_type=f32) + b1_ref[...])
    h = jax.nn.relu(lax.dot_general(h, w2_ref[...], dn,
                                    preferred_element_type=f32) + b2_ref[...])
    h = jax.nn.relu(lax.dot_general(h, w3_ref[...], dn,
                                    preferred_element_type=f32) + b3_ref[...])
    shared = lax.dot_general(h, w4_ref[...], dn,
                             preferred_element_type=f32) + b4_ref[...]
    scores_ref[...] = (lax.dot_general(shared, wu_ref[...], dn,
                                       preferred_element_type=f32)
                       + bu_ref[...] + noise_ref[...])
    logits_ref[...] = (lax.dot_general(shared, wc_ref[...], dn,
                                       preferred_element_type=f32)
                       + bc_ref[...])

    @pl.when(pl.program_id(0) == 0)
    def _():
        l2 = jnp.float32(0.0)
        for r in (w1_ref, b1_ref, w2_ref, b2_ref, w3_ref, b3_ref,
                  w4_ref, b4_ref, wu_ref, bu_ref, wc_ref, bc_ref):
            v = r[...]
            l2 = l2 + jnp.sqrt(jnp.sum(v * v))
        l2_ref[0, 0] = 0.01 * l2


def _tc_stage(x, W1, b1, W2, b2, W3, b3, W4, b4, Wu, bu, Wc, bc, noise):
    grid = BATCH // BBLK
    full = lambda i: (0, 0)
    return pl.pallas_call(
        _tc_body,
        grid=(grid,),
        in_specs=[
            pl.BlockSpec((BBLK, INCH, HW), lambda i: (i, 0, 0)),
            pl.BlockSpec((HIDDEN, INCH), full),
            pl.BlockSpec((1, HIDDEN), full),
            pl.BlockSpec((HIDDEN, HIDDEN), full),
            pl.BlockSpec((1, HIDDEN), full),
            pl.BlockSpec((HIDDEN // 2, HIDDEN), full),
            pl.BlockSpec((1, HIDDEN // 2), full),
            pl.BlockSpec((HIDDEN // 2, HIDDEN // 2), full),
            pl.BlockSpec((1, HIDDEN // 2), full),
            pl.BlockSpec((NUM_UNITS, HIDDEN // 2), full),
            pl.BlockSpec((1, NUM_UNITS), full),
            pl.BlockSpec((NUM_LABELS, HIDDEN // 2), full),
            pl.BlockSpec((1, NUM_LABELS), full),
            pl.BlockSpec((BBLK, NUM_UNITS), lambda i: (i, 0)),
        ],
        out_specs=[
            pl.BlockSpec((BBLK, NUM_UNITS), lambda i: (i, 0)),
            pl.BlockSpec((BBLK, NUM_LABELS), lambda i: (i, 0)),
            pl.BlockSpec(memory_space=pltpu.SMEM),
        ],
        out_shape=[
            jax.ShapeDtypeStruct((BATCH, NUM_UNITS), jnp.float32),
            jax.ShapeDtypeStruct((BATCH, NUM_LABELS), jnp.float32),
            jax.ShapeDtypeStruct((1, 1), jnp.float32),
        ],
    )(x, W1, b1, W2, b2, W3, b3, W4, b4, Wu, bu, Wc, bc, noise)


@functools.cache
def _build_sc_topk():
    @functools.partial(
        pl.kernel,
        mesh=plsc.VectorSubcoreMesh(core_axis_name="c", subcore_axis_name="s"),
        out_type=[
            jax.ShapeDtypeStruct((BATCH, 16), jnp.float32),
            jax.ShapeDtypeStruct((BATCH, 16), jnp.int32),
            jax.ShapeDtypeStruct((NW, NUM_UNITS), jnp.float32),
        ],
        scratch_types=[
            pltpu.VMEM((ROWS_PER_W * 4, 16), jnp.float32),
            pltpu.VMEM((ROWS_PER_W, 16), jnp.float32),
            pltpu.VMEM((ROWS_PER_W, 16), jnp.int32),
            pltpu.VMEM((NUM_UNITS,), jnp.float32),
        ],
        compiler_params=pltpu.CompilerParams(needs_layout_passes=False),
    )
    def _sc_topk(scores_hbm, probs_hbm, idx_hbm, hist_hbm, sc_v, pr_v, ix_v, h_v):
        _sc_topk_body(scores_hbm, probs_hbm, idx_hbm, hist_hbm,
                      sc_v, pr_v, ix_v, h_v)

    return _sc_topk


def _sc_topk_body(scores_hbm, probs_hbm, idx_hbm, hist_hbm, sc_v, pr_v, ix_v, h_v):
    wid = lax.axis_index("s") * 2 + lax.axis_index("c")
    base = wid * ROWS_PER_W
    pltpu.sync_copy(scores_hbm.at[pl.ds(base * 4, ROWS_PER_W * 4)], sc_v)
    zero16 = jnp.zeros((16,), jnp.float32)
    for j in range(NUM_UNITS // 16):
        h_v[pl.ds(j * 16, 16)] = zero16
    lane = lax.iota(jnp.int32, 16)
    lo8 = lane < 8
    ones16 = jnp.ones((16,), jnp.float32)

    def merge(ka, va, kb, vb):
        mk = jnp.where(lo8, ka, lax.rev(kb, (0,)))
        mv = jnp.where(lo8, va, lax.rev(vb, (0,)))
        return plsc.sort_key_val(mk, mv, descending=True)

    def row(r, carry):
        ks, vs = [], []
        for j in range(4):
            k, v = plsc.sort_key_val(sc_v[r * 4 + j], lane + j * 16,
                                     descending=True)
            ks.append(k)
            vs.append(v)
        k01, v01 = merge(ks[0], vs[0], ks[1], vs[1])
        k23, v23 = merge(ks[2], vs[2], ks[3], vs[3])
        kf, vf = merge(k01, v01, k23, v23)
        m = jnp.max(kf)
        e = jnp.where(lo8, jnp.exp(kf - m), 0.0)
        pr_v[r] = e / jnp.sum(e)
        ix_v[r] = vf
        plsc.addupdate_scatter(h_v, [vf], ones16, mask=lo8)
        return carry

    lax.fori_loop(0, ROWS_PER_W, row, 0)
    pltpu.sync_copy(pr_v, probs_hbm.at[pl.ds(base, ROWS_PER_W)])
    pltpu.sync_copy(ix_v, idx_hbm.at[pl.ds(base, ROWS_PER_W)])
    pltpu.sync_copy(h_v, hist_hbm.at[wid])


def kernel(inputs, W1, b1, W2, b2, W3, b3, W4, b4, Wu, bu, Wc, bc):
    batch = inputs.shape[0]
    x = inputs.reshape(batch, INCH, HW)
    noise = jax.random.normal(jax.random.key(42), (batch, NUM_UNITS),
                              dtype=jnp.float32) * 0.01
    scores, logits, l2 = _tc_stage(
        x, W1, b1.reshape(1, -1), W2, b2.reshape(1, -1),
        W3, b3.reshape(1, -1), W4, b4.reshape(1, -1),
        Wu, bu.reshape(1, -1), Wc, bc.reshape(1, -1), noise)
    probs_pad, idx_pad, hist = _build_sc_topk()(scores.reshape(batch * 4, 16))
    probabilities = probs_pad[:, :TOP_K]
    top_k_indices = idx_pad[:, :TOP_K]
    unit_usage = hist.sum(axis=0) * (1.0 / (batch * TOP_K))
    return (probabilities, top_k_indices, logits, l2[0, 0], unit_usage)
